# baseline (speedup 1.0000x reference)
import jax
import jax.numpy as jnp
from jax.experimental import pallas as pl
from jax.experimental.pallas import tpu as pltpu

SEQ_BLOCK = 64
LANES = 128


def _body(posx_ref, x_ref, pe_ref, tab_ref, o_ref):
    i = pl.program_id(0)
    s_blk, mid, lanes = x_ref.shape
    s_ids = i * s_blk + jax.lax.broadcasted_iota(jnp.int32, (s_blk, 1, 1), 0)
    posx = posx_ref[...]
    c = jnp.zeros((s_blk, mid, lanes), jnp.float32)
    for k in range(posx_ref.shape[0]):
        c = c + (s_ids == posx[k][None, :, :]).astype(jnp.float32)
    pe = pe_ref[...]
    tab = tab_ref[...]
    o_ref[...] = x_ref[...] + pe[:, None, :] + c * tab[:, None, :]


def kernel(x, peak_positions, pe, peak_table):
    seq_len, batch, dim = x.shape
    num_peaks = peak_positions.shape[1]
    rep = LANES // dim
    mid = batch // rep
    x3 = x.reshape(seq_len, mid, LANES)
    pe2 = jnp.tile(pe[:seq_len], (1, rep))
    tab2 = jnp.tile(peak_table[:seq_len], (1, rep))
    posx = jnp.repeat(
        peak_positions.T.reshape(num_peaks, mid, rep), dim, axis=2
    )
    grid = (seq_len // SEQ_BLOCK,)
    out3 = pl.pallas_call(
        _body,
        grid=grid,
        in_specs=[
            pl.BlockSpec((num_peaks, mid, LANES), lambda i: (0, 0, 0)),
            pl.BlockSpec((SEQ_BLOCK, mid, LANES), lambda i: (i, 0, 0)),
            pl.BlockSpec((SEQ_BLOCK, LANES), lambda i: (i, 0)),
            pl.BlockSpec((SEQ_BLOCK, LANES), lambda i: (i, 0)),
        ],
        out_specs=pl.BlockSpec((SEQ_BLOCK, mid, LANES), lambda i: (i, 0, 0)),
        out_shape=jax.ShapeDtypeStruct(x3.shape, x.dtype),
        compiler_params=pltpu.CompilerParams(
            dimension_semantics=("parallel",),
        ),
    )(posx, x3, pe2, tab2)
    return out3.reshape(seq_len, batch, dim)
